# Initial kernel scaffold; baseline (speedup 1.0000x reference)
import jax
import jax.numpy as jnp
from jax.experimental import pallas as pl


def kernel(input_ids, word_emb, pos_emb, type_emb, gamma, beta):
    raise NotImplementedError("write your pallas kernel here")


# speedup vs baseline: 3.2272x; 3.2272x over previous
import jax
import jax.numpy as jnp
from jax import lax
from jax.experimental import pallas as pl
from jax.experimental.pallas import tpu as pltpu
from jax.experimental.pallas import tpu_sc as plsc

B = 4096
S = 200
H = 128
N = B * S
NC, NS, L = 2, 16, 16
NW = NC * NS
ROWS_W = N // NW
CH = 128
NCH = ROWS_W // CH
SEG = H // L
EPS = 1e-12


def _rsqrt(x):
    i = lax.bitcast_convert_type(x, jnp.int32)
    i = jnp.int32(0x5F3759DF) - lax.shift_right_logical(i, 1)
    y = lax.bitcast_convert_type(i, jnp.float32)
    for _ in range(3):
        y = y * (1.5 - 0.5 * x * y * y)
    return y


def _body(ids_hbm, wemb_hbm, pemb_hbm, temb_hbm, gamma_hbm, beta_hbm,
          out_hbm, idx_v, rows_v, pe_v, aux_v, sem):
    wid = lax.axis_index("s") * NC + lax.axis_index("c")

    pltpu.sync_copy(pemb_hbm.at[pl.ds(0, S)], pe_v)
    pltpu.sync_copy(temb_hbm.at[pl.ds(0, 1)], aux_v.at[pl.ds(0, 1)])
    pltpu.sync_copy(gamma_hbm, aux_v.at[1])
    pltpu.sync_copy(beta_hbm, aux_v.at[2])

    def fold(r, carry):
        for k in range(SEG):
            sl = pl.ds(k * L, L)
            pe_v[r, sl] = pe_v[r, sl] + aux_v[0, sl]
        return carry
    lax.fori_loop(0, S, fold, 0)

    def chunk(c, carry):
        base = wid * ROWS_W + c * CH
        pltpu.sync_copy(ids_hbm.at[pl.ds(base, CH)], idx_v)
        pltpu.async_copy(wemb_hbm.at[idx_v], rows_v, sem).wait()

        def row(r, rcarry):
            pos = lax.rem(base + r, S)
            y = []
            for k in range(SEG):
                sl = pl.ds(k * L, L)
                y.append(rows_v[r, sl] + pe_v[pos, sl])
            s = y[0]
            q = y[0] * y[0]
            for k in range(1, SEG):
                s = s + y[k]
                q = q + y[k] * y[k]
            tot = jnp.sum(s)
            tot2 = jnp.sum(q)
            mean = tot * (1.0 / H)
            var = tot2 * (1.0 / H) - mean * mean
            inv = _rsqrt(var + EPS)
            for k in range(SEG):
                sl = pl.ds(k * L, L)
                rows_v[r, sl] = ((y[k] - mean) * inv) * aux_v[1, sl] \
                    + aux_v[2, sl]
            return rcarry
        lax.fori_loop(0, CH, row, 0)

        pltpu.sync_copy(rows_v, out_hbm.at[pl.ds(base, CH)])
        return carry
    lax.fori_loop(0, NCH, chunk, 0)


def kernel(input_ids, word_emb, pos_emb, type_emb, gamma, beta):
    ids = input_ids.reshape(-1).astype(jnp.int32)
    mesh = plsc.VectorSubcoreMesh(
        core_axis_name="c", subcore_axis_name="s",
        num_cores=NC, num_subcores=NS)
    out = pl.kernel(
        _body,
        out_type=jax.ShapeDtypeStruct((N, H), jnp.float32),
        mesh=mesh,
        compiler_params=pltpu.CompilerParams(needs_layout_passes=False),
        scratch_types=[
            pltpu.VMEM((CH,), jnp.int32),
            pltpu.VMEM((CH, H), jnp.float32),
            pltpu.VMEM((S, H), jnp.float32),
            pltpu.VMEM((3, H), jnp.float32),
            pltpu.SemaphoreType.DMA,
        ],
    )(ids, word_emb, pos_emb, type_emb, gamma, beta)
    return out.reshape(B, S, H)


# speedup vs baseline: 5.6069x; 1.7374x over previous
import jax
import jax.numpy as jnp
from jax import lax
from jax.experimental import pallas as pl
from jax.experimental.pallas import tpu as pltpu
from jax.experimental.pallas import tpu_sc as plsc

B = 4096
S = 200
H = 128
N = B * S
NC, NS, L = 2, 16, 16
NW = NC * NS
ROWS_W = N // NW
CH = 128
NCH = ROWS_W // CH
NBUF = 4
SEG = H // L
UNROLL = 4
EPS = 1e-12


def _rsqrt(x):
    i = lax.bitcast_convert_type(x, jnp.int32)
    i = jnp.int32(0x5F3759DF) - lax.shift_right_logical(i, 1)
    y = lax.bitcast_convert_type(i, jnp.float32)
    for _ in range(3):
        y = y * (1.5 - 0.5 * x * y * y)
    return y


def _body(ids_hbm, wemb_hbm, pemb_hbm, temb_hbm, gamma_hbm, beta_hbm,
          out_hbm, idx_v, pe_v, aux_v, bufs, gsems, osems):
    wid = lax.axis_index("s") * NC + lax.axis_index("c")
    row0 = wid * ROWS_W

    pltpu.sync_copy(ids_hbm.at[wid], idx_v)
    pltpu.sync_copy(pemb_hbm.at[pl.ds(0, S)], pe_v)
    pltpu.sync_copy(temb_hbm.at[pl.ds(0, 1)], aux_v.at[pl.ds(0, 1)])
    pltpu.sync_copy(gamma_hbm, aux_v.at[1])
    pltpu.sync_copy(beta_hbm, aux_v.at[2])

    def fold(r, carry):
        for k in range(SEG):
            sl = pl.ds(k * L, L)
            pe_v[r, sl] = pe_v[r, sl] + aux_v[0, sl]
        return carry
    lax.fori_loop(0, S, fold, 0)

    def gather(b, c, wait=False):
        mk = pltpu.make_async_copy if wait else pltpu.async_copy
        cp = mk(wemb_hbm.at[idx_v.at[c]], bufs[b], gsems[b])
        if wait:
            cp.wait()
        return cp

    def out_copy(b, c, wait=False):
        dst = out_hbm.at[pl.ds(row0 + c * CH, CH)]
        mk = pltpu.make_async_copy if wait else pltpu.async_copy
        cp = mk(bufs[b], dst, osems[b])
        if wait:
            cp.wait()
        return cp

    for b in range(NBUF):
        gather(b, b)

    gb = tuple(aux_v[1, pl.ds(k * L, L)] for k in range(SEG)) \
        + tuple(aux_v[2, pl.ds(k * L, L)] for k in range(SEG))

    def one_row(buf, r, pos0, gbv):
        p = pos0 + r
        p = jnp.where(p >= S, p - S, p)
        y = [buf[r, pl.ds(k * L, L)] + pe_v[p, pl.ds(k * L, L)]
             for k in range(SEG)]
        sa = (y[0] + y[1]) + (y[2] + y[3])
        sb = (y[4] + y[5]) + (y[6] + y[7])
        qa = (y[0] * y[0] + y[1] * y[1]) + (y[2] * y[2] + y[3] * y[3])
        qb = (y[4] * y[4] + y[5] * y[5]) + (y[6] * y[6] + y[7] * y[7])
        tot = jnp.sum(sa + sb)
        tot2 = jnp.sum(qa + qb)
        mean = tot * (1.0 / H)
        var = tot2 * (1.0 / H) - mean * mean
        inv = _rsqrt(var + EPS)
        m2 = mean * inv
        for k in range(SEG):
            buf[r, pl.ds(k * L, L)] = \
                (y[k] * inv - m2) * gbv[k] + gbv[SEG + k]

    def compute_chunk(buf, c, gbv):
        pos0 = lax.rem(c * CH, S)

        def rows(g, carry):
            base_r = g * UNROLL
            for u in range(UNROLL):
                one_row(buf, base_r + u, pos0, carry)
            return carry
        lax.fori_loop(0, CH // UNROLL, rows, gbv)

    def step(s, carry):
        for b in range(NBUF):
            c = s * NBUF + b
            gather(b, c, wait=True)
            compute_chunk(bufs[b], c, carry)
            out_copy(b, c)
        for b in range(NBUF):
            c = s * NBUF + b
            out_copy(b, c, wait=True)

            @pl.when(s < NCH // NBUF - 1)
            def _():
                gather(b, c + NBUF)
        return carry
    lax.fori_loop(0, NCH // NBUF, step, gb)


def kernel(input_ids, word_emb, pos_emb, type_emb, gamma, beta):
    ids = input_ids.reshape(NW, NCH, CH).astype(jnp.int32)
    mesh = plsc.VectorSubcoreMesh(
        core_axis_name="c", subcore_axis_name="s",
        num_cores=NC, num_subcores=NS)
    out = pl.kernel(
        _body,
        out_type=jax.ShapeDtypeStruct((N, H), jnp.float32),
        mesh=mesh,
        compiler_params=pltpu.CompilerParams(needs_layout_passes=False),
        scratch_types=[
            pltpu.VMEM((NCH, CH), jnp.int32),
            pltpu.VMEM((S, H), jnp.float32),
            pltpu.VMEM((3, H), jnp.float32),
            [pltpu.VMEM((CH, H), jnp.float32) for _ in range(NBUF)],
            [pltpu.SemaphoreType.DMA for _ in range(NBUF)],
            [pltpu.SemaphoreType.DMA for _ in range(NBUF)],
        ],
    )(ids, word_emb, pos_emb, type_emb, gamma, beta)
    return out.reshape(B, S, H)


# speedup vs baseline: 7.0394x; 1.2555x over previous
import jax
import jax.numpy as jnp
from jax import lax
from jax.experimental import pallas as pl
from jax.experimental.pallas import tpu as pltpu
from jax.experimental.pallas import tpu_sc as plsc

B = 4096
S = 200
H = 128
N = B * S
NC, NS, L = 2, 16, 16
NW = NC * NS
ROWS_W = N // NW
CH = 128
NCH = ROWS_W // CH
NBUF = 4
SEG = H // L
EPS = 1e-12


def _sl(k):
    return pl.ds(k * L, L)


def _vrsqrt(x):
    i = plsc.bitcast(x, jnp.int32)
    i = jnp.int32(0x5F3759DF) - lax.shift_right_logical(i, 1)
    y = plsc.bitcast(i, jnp.float32)
    for _ in range(3):
        y = y * (1.5 - 0.5 * x * y * y)
    return y


def _body(ids_hbm, wemb_hbm, pemb_hbm, temb_hbm, gamma_hbm, beta_hbm,
          out_hbm, idx_v, pe_v, aux_v, ybuf, sbuf, ivbuf, bufs,
          gsems, osems):
    wid = lax.axis_index("s") * NC + lax.axis_index("c")
    row0 = wid * ROWS_W

    pltpu.sync_copy(ids_hbm.at[wid], idx_v)
    pltpu.sync_copy(pemb_hbm.at[pl.ds(0, S)], pe_v)
    pltpu.sync_copy(temb_hbm.at[pl.ds(0, 1)], aux_v.at[pl.ds(0, 1)])
    pltpu.sync_copy(gamma_hbm, aux_v.at[1])
    pltpu.sync_copy(beta_hbm, aux_v.at[2])

    def fold(r, carry):
        for k in range(SEG):
            pe_v[r, _sl(k)] = pe_v[r, _sl(k)] + aux_v[0, _sl(k)]
        return carry
    lax.fori_loop(0, S, fold, 0)

    def gather(b, c, wait=False):
        mk = pltpu.make_async_copy if wait else pltpu.async_copy
        cp = mk(wemb_hbm.at[idx_v.at[c]], bufs[b], gsems[b])
        if wait:
            cp.wait()
        return cp

    def out_copy(b, c, wait=False):
        dst = out_hbm.at[pl.ds(row0 + c * CH, CH)]
        mk = pltpu.make_async_copy if wait else pltpu.async_copy
        cp = mk(bufs[b], dst, osems[b])
        if wait:
            cp.wait()
        return cp

    for b in range(NBUF):
        gather(b, b)

    gb = tuple(aux_v[1, _sl(k)] for k in range(SEG)) \
        + tuple(aux_v[2, _sl(k)] for k in range(SEG))

    iota = lax.iota(jnp.int32, L)

    def compute_chunk(buf, c, gbv):
        pos0 = lax.rem(c * CH, S)

        def group(g, carry):
            r0 = g * L
            for u in range(L):
                r = r0 + u
                p = pos0 + r
                p = jnp.where(p >= S, p - S, p)
                xs = [buf[r, _sl(k)] for k in range(SEG)]
                ps = [pe_v[p, _sl(k)] for k in range(SEG)]
                y = [xs[k] + ps[k] for k in range(SEG)]
                for k in range(SEG):
                    ybuf[u, _sl(k)] = y[k]
                sa = (y[0] + y[1]) + (y[2] + y[3])
                sb = (y[4] + y[5]) + (y[6] + y[7])
                qa = (y[0] * y[0] + y[1] * y[1]) \
                    + (y[2] * y[2] + y[3] * y[3])
                qb = (y[4] * y[4] + y[5] * y[5]) \
                    + (y[6] * y[6] + y[7] * y[7])
                sbuf[pl.ds(u * 2 * L, L)] = sa + sb
                sbuf[pl.ds(u * 2 * L + L, L)] = qa + qb
            base32 = iota * (2 * L)
            gs = [plsc.load_gather(sbuf, [base32 + j]) for j in range(L)]
            qs = [plsc.load_gather(sbuf, [base32 + (L + j)])
                  for j in range(L)]
            tot = (((gs[0] + gs[1]) + (gs[2] + gs[3]))
                   + ((gs[4] + gs[5]) + (gs[6] + gs[7]))) \
                + (((gs[8] + gs[9]) + (gs[10] + gs[11]))
                   + ((gs[12] + gs[13]) + (gs[14] + gs[15])))
            tot2 = (((qs[0] + qs[1]) + (qs[2] + qs[3]))
                    + ((qs[4] + qs[5]) + (qs[6] + qs[7]))) \
                + (((qs[8] + qs[9]) + (qs[10] + qs[11]))
                   + ((qs[12] + qs[13]) + (qs[14] + qs[15])))
            mean = tot * (1.0 / H)
            var = tot2 * (1.0 / H) - mean * mean
            inv = _vrsqrt(var + EPS)
            m2 = mean * inv
            ivbuf[pl.ds(0, L)] = inv
            ivbuf[pl.ds(L, L)] = m2
            for u2 in range(0, L, 2):
                ivs, mvs, yss = [], [], []
                for u in (u2, u2 + 1):
                    uu = jnp.full((L,), u, jnp.int32)
                    ivs.append(plsc.load_gather(ivbuf, [uu]))
                    mvs.append(plsc.load_gather(ivbuf, [uu + L]))
                for u in (u2, u2 + 1):
                    yss.append([ybuf[u, _sl(k)] for k in range(SEG)])
                for i, u in enumerate((u2, u2 + 1)):
                    outs = [(yss[i][k] * ivs[i] - mvs[i]) * gbv[k]
                            + gbv[SEG + k] for k in range(SEG)]
                    for k in range(SEG):
                        buf[r0 + u, _sl(k)] = outs[k]
            return carry
        lax.fori_loop(0, CH // L, group, gbv)

    def step(s, carry):
        for b in range(NBUF):
            c = s * NBUF + b
            gather(b, c, wait=True)
            compute_chunk(bufs[b], c, carry)
            out_copy(b, c)
        for b in range(NBUF):
            c = s * NBUF + b
            out_copy(b, c, wait=True)

            @pl.when(s < NCH // NBUF - 1)
            def _():
                gather(b, c + NBUF)
        return carry
    lax.fori_loop(0, NCH // NBUF, step, gb)


def kernel(input_ids, word_emb, pos_emb, type_emb, gamma, beta):
    ids = input_ids.reshape(NW, NCH, CH).astype(jnp.int32)
    mesh = plsc.VectorSubcoreMesh(
        core_axis_name="c", subcore_axis_name="s",
        num_cores=NC, num_subcores=NS)
    out = pl.kernel(
        _body,
        out_type=jax.ShapeDtypeStruct((N, H), jnp.float32),
        mesh=mesh,
        compiler_params=pltpu.CompilerParams(needs_layout_passes=False),
        scratch_types=[
            pltpu.VMEM((NCH, CH), jnp.int32),
            pltpu.VMEM((S, H), jnp.float32),
            pltpu.VMEM((3, H), jnp.float32),
            pltpu.VMEM((L, H), jnp.float32),
            pltpu.VMEM((2 * L * L,), jnp.float32),
            pltpu.VMEM((2 * L,), jnp.float32),
            [pltpu.VMEM((CH, H), jnp.float32) for _ in range(NBUF)],
            [pltpu.SemaphoreType.DMA for _ in range(NBUF)],
            [pltpu.SemaphoreType.DMA for _ in range(NBUF)],
        ],
    )(ids, word_emb, pos_emb, type_emb, gamma, beta)
    return out.reshape(B, S, H)
